# baseline (speedup 1.0000x reference)
import functools
import jax
import jax.numpy as jnp
from jax import lax
from jax.experimental import pallas as pl
from jax.experimental.pallas import tpu as pltpu
from jax.experimental.pallas import tpu_sc as plsc

NC, NS, LANES = 2, 16, 16
NW = NC * NS
D = 64
CW = 512


def build(B):
    bpw = B // NW
    nwin = bpw // CW
    mesh = plsc.VectorSubcoreMesh(core_axis_name="c", subcore_axis_name="s")

    @functools.partial(
        pl.kernel,
        out_type=jax.ShapeDtypeStruct((B, D), jnp.float32),
        mesh=mesh,
        scratch_types=[
            pltpu.VMEM((bpw,), jnp.int32),
            pltpu.VMEM((CW, D), jnp.float32),
            pltpu.VMEM_SHARED((NS, CW, D), jnp.float32),
            pltpu.SemaphoreType.DMA,
            pltpu.SemaphoreType.DMA,
        ],
        compiler_params=pltpu.CompilerParams(use_tc_tiling_on_sc=False),
    )
    def k(x_hbm, emb_hbm, out_hbm, idx_all, rv, sp, gsem, csem):
        wid = lax.axis_index("s") * NC + lax.axis_index("c")
        sid = lax.axis_index("s")
        base = wid * bpw
        pltpu.sync_copy(x_hbm.at[pl.ds(base, bpw)], idx_all)

        def window(g, carry):
            woff = g * CW

            def fire(j16, c2):
                iv = idx_all[pl.ds(woff + j16 * LANES, LANES)]
                for r in range(LANES):
                    s = iv[r]
                    pltpu.async_copy(
                        emb_hbm.at[pl.ds(s, 1)],
                        sp.at[sid, pl.ds(j16 * LANES + r, 1)], gsem)
                return c2

            lax.fori_loop(0, CW // LANES, fire, 0)
            pltpu.make_async_copy(out_hbm.at[pl.ds(0, CW)], sp.at[sid],
                                  gsem).wait()
            pltpu.sync_copy(sp.at[sid], rv)

            @plsc.parallel_loop(0, CW, step=1, unroll=8)
            def _scale(i):
                for kk in range(D // LANES):
                    sl = pl.ds(kk * LANES, LANES)
                    rv[i, sl] = rv[i, sl] * 8.0

            pltpu.sync_copy(rv, out_hbm.at[pl.ds(base + woff, CW)])
            return carry

        lax.fori_loop(0, nwin, window, 0)

    return k


def kernel(x, emb):
    s0, s1 = x.shape
    xf = x.reshape(-1).astype(jnp.int32)
    out = build(s0 * s1)(xf, emb)
    return out.reshape(s0, s1, D)


# speedup vs baseline: 1.7507x; 1.7507x over previous
import functools

import jax
import jax.numpy as jnp
from jax import lax
from jax.experimental import pallas as pl
from jax.experimental.pallas import tpu as pltpu
from jax.experimental.pallas import tpu_sc as plsc

NC, NS, LANES = 2, 16, 16
NW = NC * NS
D = 64
SCALE = 8.0
CW = 512
DROWS = 144
SROWS = CW - DROWS


@functools.lru_cache(maxsize=None)
def _build(B: int):
    assert B % (NW * CW) == 0, B
    bpw = B // NW
    nwin = bpw // CW
    mesh = plsc.VectorSubcoreMesh(core_axis_name="c", subcore_axis_name="s")

    @functools.partial(
        pl.kernel,
        out_type=jax.ShapeDtypeStruct((B, D), jnp.float32),
        mesh=mesh,
        scratch_types=[
            pltpu.VMEM((bpw,), jnp.int32),
            pltpu.VMEM((CW, D), jnp.float32),
            pltpu.VMEM_SHARED((NS, DROWS, D), jnp.float32),
            pltpu.SemaphoreType.DMA,
            pltpu.SemaphoreType.DMA,
        ],
        compiler_params=pltpu.CompilerParams(use_tc_tiling_on_sc=False),
    )
    def emb_kernel(x_hbm, emb_hbm, out_hbm, idx_all, rv, sp, gsem, csem):
        wid = lax.axis_index("s") * NC + lax.axis_index("c")
        sid = lax.axis_index("s")
        base = wid * bpw
        pltpu.sync_copy(x_hbm.at[pl.ds(base, bpw)], idx_all)

        def window(g, carry):
            woff = g * CW
            pltpu.async_copy(
                emb_hbm.at[idx_all.at[pl.ds(woff, SROWS)]],
                rv.at[pl.ds(0, SROWS)], gsem)

            def fire(j16, c2):
                iv = idx_all[pl.ds(woff + SROWS + j16 * LANES, LANES)]
                for r in range(LANES):
                    s = iv[r]
                    pltpu.async_copy(
                        emb_hbm.at[pl.ds(s, 1)],
                        sp.at[sid, pl.ds(j16 * LANES + r, 1)], csem)
                return c2

            lax.fori_loop(0, DROWS // LANES, fire, 0)

            pltpu.make_async_copy(out_hbm.at[pl.ds(0, DROWS)], sp.at[sid],
                                  csem).wait()
            pltpu.sync_copy(sp.at[sid], rv.at[pl.ds(SROWS, DROWS)])
            pltpu.make_async_copy(out_hbm.at[pl.ds(0, SROWS)],
                                  rv.at[pl.ds(0, SROWS)], gsem).wait()

            @plsc.parallel_loop(0, CW, step=1, unroll=8)
            def _scale(i):
                for k in range(D // LANES):
                    sl = pl.ds(k * LANES, LANES)
                    rv[i, sl] = rv[i, sl] * SCALE

            pltpu.sync_copy(rv, out_hbm.at[pl.ds(base + woff, CW)])
            return carry

        lax.fori_loop(0, nwin, window, 0)

    return emb_kernel


def kernel(x, emb):
    s0, s1 = x.shape
    xf = x.reshape(-1).astype(jnp.int32)
    out = _build(s0 * s1)(xf, emb)
    return out.reshape(s0, s1, D)


# speedup vs baseline: 2.1850x; 1.2481x over previous
import functools

import jax
import jax.numpy as jnp
from jax import lax
from jax.experimental import pallas as pl
from jax.experimental.pallas import tpu as pltpu
from jax.experimental.pallas import tpu_sc as plsc

NC, NS, LANES = 2, 16, 16
NW = NC * NS
D = 64
SCALE = 8.0
CHUNK = 400
NBUF = 4


@functools.lru_cache(maxsize=None)
def _build(B: int):
    assert B % (NW * CHUNK) == 0, B
    bpw = B // NW
    nchunk = bpw // CHUNK
    mesh = plsc.VectorSubcoreMesh(core_axis_name="c", subcore_axis_name="s")

    @functools.partial(
        pl.kernel,
        out_type=jax.ShapeDtypeStruct((B, D), jnp.float32),
        mesh=mesh,
        scratch_types=[
            pltpu.VMEM((bpw,), jnp.int32),
            [pltpu.VMEM((CHUNK, D), jnp.float32) for _ in range(NBUF)],
            [pltpu.SemaphoreType.DMA for _ in range(NBUF)],
            [pltpu.SemaphoreType.DMA for _ in range(NBUF)],
        ],
        compiler_params=pltpu.CompilerParams(use_tc_tiling_on_sc=False),
    )
    def emb_kernel(x_hbm, emb_hbm, out_hbm, idx_all, rows, gsem, ssem):
        wid = lax.axis_index("s") * NC + lax.axis_index("c")
        base = wid * bpw
        pltpu.sync_copy(x_hbm.at[pl.ds(base, bpw)], idx_all)

        def start_gather(g):
            b = g % NBUF
            return pltpu.async_copy(
                emb_hbm.at[idx_all.at[pl.ds(g * CHUNK, CHUNK)]], rows[b],
                gsem[b])

        gathers = {}
        stores = {}
        for h in range(min(NBUF - 1, nchunk)):
            gathers[h] = start_gather(h)
        for g in range(nchunk):
            b = g % NBUF
            h = g + NBUF - 1
            if h < nchunk:
                hb = h % NBUF
                if hb in stores:
                    stores.pop(hb).wait()
                gathers[h] = start_gather(h)
            gathers.pop(g).wait()

            rv = rows[b]

            @plsc.parallel_loop(0, CHUNK, step=1, unroll=8)
            def _scale(i):
                for k in range(D // LANES):
                    sl = pl.ds(k * LANES, LANES)
                    rv[i, sl] = rv[i, sl] * SCALE

            off = base + g * CHUNK
            stores[b] = pltpu.async_copy(rv, out_hbm.at[pl.ds(off, CHUNK)],
                                         ssem[b])
        for b in list(stores):
            stores.pop(b).wait()

    return emb_kernel


def kernel(x, emb):
    s0, s1 = x.shape
    xf = x.reshape(-1).astype(jnp.int32)
    out = _build(s0 * s1)(xf, emb)
    return out.reshape(s0, s1, D)
